# baseline (speedup 1.0000x reference)
import jax
import jax.numpy as jnp
from jax.experimental import pallas as pl

_ROWS = 128
_COLS = 100000
_ROW_BLOCK = 16
_SUP_STRIDE = 200
_SUP_LIMIT = 6400


def _mask_body(x_ref, o_ref):
    col = jax.lax.broadcasted_iota(jnp.int32, x_ref.shape, 1)
    suppressed = (col < _SUP_LIMIT) & (col % _SUP_STRIDE == 0)
    o_ref[...] = jnp.where(suppressed, -jnp.inf, x_ref[...])


def kernel(scores):
    return pl.pallas_call(
        _mask_body,
        grid=(_ROWS // _ROW_BLOCK,),
        in_specs=[pl.BlockSpec((_ROW_BLOCK, _COLS), lambda i: (i, 0))],
        out_specs=pl.BlockSpec((_ROW_BLOCK, _COLS), lambda i: (i, 0)),
        out_shape=jax.ShapeDtypeStruct((_ROWS, _COLS), scores.dtype),
    )(scores)
